# baseline (speedup 1.0000x reference)
import functools

import jax
import jax.numpy as jnp
from jax import lax
from jax.experimental import pallas as pl
from jax.experimental.pallas import tpu as pltpu
from jax.experimental.pallas import tpu_sc as plsc

_DIM = 32
_BATCH = 4096
_SEQ = 200

_info = plsc.get_sparse_core_info()
_NC = _info.num_cores
_NS = _info.num_subcores
_NW = _NC * _NS
_BL = _BATCH // _NW
_SC = 5
_NCH = _SEQ // _SC

_mesh = plsc.VectorSubcoreMesh(core_axis_name="c", subcore_axis_name="s")


@functools.partial(
    pl.kernel,
    mesh=_mesh,
    out_type=jax.ShapeDtypeStruct((_SEQ, _DIM // 8, _NW, 8, _BL), jnp.float32),
    scratch_types=[
        pltpu.VMEM((_SEQ, _BL), jnp.int32),
        pltpu.VMEM((_SC * _BL, _DIM), jnp.float32),
        pltpu.VMEM((_SC * _BL, _DIM), jnp.float32),
        pltpu.VMEM((_SC, _DIM // 8, 1, 8, _BL), jnp.float32),
        pltpu.VMEM((_SC, _DIM // 8, 1, 8, _BL), jnp.float32),
        pltpu.SemaphoreType.DMA,
        pltpu.SemaphoreType.DMA,
        pltpu.SemaphoreType.DMA,
        pltpu.SemaphoreType.DMA,
    ],
    compiler_params=pltpu.CompilerParams(use_tc_tiling_on_sc=False, needs_layout_passes=False),
)
def _gather_kernel(idxT_hbm, table_hbm, out_hbm, idxT_v, r0, r1, t0, t1,
                   g0, g1, o0, o1):
    rows = (r0, r1)
    tiles = (t0, t1)
    gsem = (g0, g1)
    osem = (o0, o1)
    wid = lax.axis_index("s") * _NC + lax.axis_index("c")
    iota = lax.iota(jnp.int32, 16)
    zero16 = iota * 0

    pltpu.sync_copy(idxT_hbm.at[:, pl.ds(wid * _BL, _BL)], idxT_v)

    def fire_gathers(c, buf):
        for j in range(_SC):
            pltpu.async_copy(
                table_hbm.at[idxT_v.at[c * _SC + j]],
                rows[buf].at[pl.ds(j * _BL, _BL)], gsem[buf])

    def wait_gathers(c, buf):
        for j in range(_SC):
            pltpu.make_async_copy(
                table_hbm.at[idxT_v.at[c * _SC + j]],
                rows[buf].at[pl.ds(j * _BL, _BL)], gsem[buf]).wait()

    def out_desc(c, buf):
        return pltpu.make_async_copy(
            tiles[buf],
            out_hbm.at[pl.ds(c * _SC, _SC), pl.ds(0, _DIM // 8),
                       pl.ds(wid, 1)],
            osem[buf])

    def transpose(buf):
        @plsc.parallel_loop(0, _SC * _DIM, 1, unroll=4)
        def _body(m):
            j = m >> 5
            d = m & (_DIM - 1)
            j128 = j * _BL
            col = zero16 + d
            for g in range(_BL // 16):
                v = plsc.load_gather(rows[buf], [iota + (j128 + g * 16), col])
                tiles[buf][j, d // 8, 0, d % 8, pl.ds(g * 16, 16)] = v

    fire_gathers(0, 0)

    def slot(c, buf):
        @pl.when(c + 1 < _NCH)
        def _():
            fire_gathers(c + 1, 1 - buf)

        wait_gathers(c, buf)

        @pl.when(c >= 2)
        def _():
            out_desc(c - 2, buf).wait()

        transpose(buf)
        out_desc(c, buf).start()

    def outer(o, carry):
        slot(2 * o, 0)
        slot(2 * o + 1, 1)
        return carry

    lax.fori_loop(0, _NCH // 2, outer, 0)
    out_desc(_NCH - 2, 0).wait()
    out_desc(_NCH - 1, 1).wait()


def kernel(image, table):
    idxT = image.T.astype(jnp.int32)
    out5 = _gather_kernel(idxT, table)
    return out5.transpose(2, 4, 0, 1, 3).reshape(_BATCH, _SEQ, _DIM)


# speedup vs baseline: 1.4260x; 1.4260x over previous
import functools

import jax
import jax.numpy as jnp
from jax import lax
from jax.experimental import pallas as pl
from jax.experimental.pallas import tpu as pltpu
from jax.experimental.pallas import tpu_sc as plsc

_DIM = 32
_BATCH = 4096
_SEQ = 200

_info = plsc.get_sparse_core_info()
_NC = _info.num_cores
_NS = _info.num_subcores
_NW = _NC * _NS
_BL = _BATCH // _NW
_SC = 4
_NCH = _SEQ // _SC
_PITCH = _BL + 1

_mesh = plsc.VectorSubcoreMesh(core_axis_name="c", subcore_axis_name="s")


@functools.partial(
    pl.kernel,
    mesh=_mesh,
    out_type=jax.ShapeDtypeStruct((_SEQ, _DIM // 8, _NW, 8, _BL), jnp.float32),
    scratch_types=[
        pltpu.VMEM((_SEQ, _BL), jnp.int32),
        pltpu.VMEM((_SC * _BL, _DIM), jnp.float32),
        pltpu.VMEM((_SC * _BL, _DIM), jnp.float32),
        pltpu.VMEM((_SC, _DIM // 8, 1, 8, _PITCH), jnp.float32),
        pltpu.VMEM((_SC, _DIM // 8, 1, 8, _PITCH), jnp.float32),
        pltpu.SemaphoreType.DMA,
        pltpu.SemaphoreType.DMA,
        pltpu.SemaphoreType.DMA,
        pltpu.SemaphoreType.DMA,
    ],
    compiler_params=pltpu.CompilerParams(use_tc_tiling_on_sc=False, needs_layout_passes=False),
)
def _gather_kernel(idxT_hbm, table_hbm, out_hbm, idxT_v, r0, r1, t0, t1,
                   g0, g1, o0, o1):
    rows = (r0, r1)
    tiles = (t0, t1)
    gsem = (g0, g1)
    osem = (o0, o1)
    wid = lax.axis_index("s") * _NC + lax.axis_index("c")
    iota = lax.iota(jnp.int32, 16)
    zero16 = iota * 0

    pltpu.sync_copy(idxT_hbm.at[:, pl.ds(wid * _BL, _BL)], idxT_v)

    def fire_gathers(c, buf):
        for j in range(_SC):
            pltpu.async_copy(
                table_hbm.at[idxT_v.at[c * _SC + j]],
                rows[buf].at[pl.ds(j * _BL, _BL)], gsem[buf])

    def wait_gathers(c, buf):
        for j in range(_SC):
            pltpu.make_async_copy(
                table_hbm.at[idxT_v.at[c * _SC + j]],
                rows[buf].at[pl.ds(j * _BL, _BL)], gsem[buf]).wait()

    def out_desc(c, buf):
        return pltpu.make_async_copy(
            tiles[buf].at[:, :, :, :, pl.ds(0, _BL)],
            out_hbm.at[pl.ds(c * _SC, _SC), pl.ds(0, _DIM // 8),
                       pl.ds(wid, 1)],
            osem[buf])

    dv = (iota >> 3, (iota >> 3) + 2)
    rv = (iota & 7, iota & 7)

    def transpose(buf):
        @plsc.parallel_loop(0, _BL, 1, unroll=2)
        def _body(l):
            l_vec = zero16 + l
            for j in range(_SC):
                j_vec = zero16 + j
                row = j * _BL + l
                for h in range(2):
                    v = rows[buf][row, pl.ds(h * 16, 16)]
                    plsc.store_scatter(
                        tiles[buf], [j_vec, dv[h], zero16, rv[h], l_vec], v)

    fire_gathers(0, 0)

    def slot(c, buf):
        @pl.when(c + 1 < _NCH)
        def _():
            fire_gathers(c + 1, 1 - buf)

        wait_gathers(c, buf)

        @pl.when(c >= 2)
        def _():
            out_desc(c - 2, buf).wait()

        transpose(buf)
        out_desc(c, buf).start()

    def outer(o, carry):
        slot(2 * o, 0)
        slot(2 * o + 1, 1)
        return carry

    lax.fori_loop(0, _NCH // 2, outer, 0)
    out_desc(_NCH - 2, 0).wait()
    out_desc(_NCH - 1, 1).wait()


def kernel(image, table):
    idxT = image.T.astype(jnp.int32)
    out5 = _gather_kernel(idxT, table)
    return out5.transpose(2, 4, 0, 1, 3).reshape(_BATCH, _SEQ, _DIM)
